# Initial kernel scaffold; baseline (speedup 1.0000x reference)
import jax
import jax.numpy as jnp
from jax.experimental import pallas as pl


def kernel(x, edge_index, Wl0, bl0, Wr0, Wl1, bl1, Wr1, Wl2, bl2, Wr2):
    raise NotImplementedError("write your pallas kernel here")


# speedup vs baseline: 3.3487x; 3.3487x over previous
import functools

import jax
import jax.numpy as jnp
from jax import lax
from jax.experimental import pallas as pl
from jax.experimental.pallas import tpu as pltpu
from jax.experimental.pallas import tpu_sc as plsc

N = 10000
E = 160000
D_IN = 256
D_HID = 128
D_OUT = 64

NC = 2
NS = 16
NW = NC * NS
CHUNK = 128
NCH = (E + NW * CHUNK - 1) // (NW * CHUNK)
EPAD = NW * NCH * CHUNK
NP = 10240
RPT = NP // NS


def _sc_body(with_deg, d, table, srcs, dsts, zrows, zdeg, ones_in,
             out_acc, out_deg, src_v, dst_v, rows_v, ones_v, acc_sh,
             deg_sh, sem):
    c = lax.axis_index("c")
    s = lax.axis_index("s")
    wid = s * NC + c

    pltpu.sync_copy(zrows, acc_sh.at[pl.ds(s * RPT, RPT)])
    pltpu.sync_copy(srcs.at[wid], src_v)
    pltpu.sync_copy(dsts.at[wid], dst_v)
    if with_deg:
        pltpu.sync_copy(zdeg, deg_sh.at[pl.ds(s * RPT, RPT)])
        pltpu.sync_copy(ones_in, ones_v)
    plsc.subcore_barrier()

    def body(j, carry):
        pltpu.async_copy(table.at[src_v.at[j]], rows_v, sem).wait()
        pltpu.sync_copy(rows_v, acc_sh.at[dst_v.at[j]], add=True)
        if with_deg:
            pltpu.sync_copy(ones_v, deg_sh.at[dst_v.at[j]], add=True)
        return carry

    lax.fori_loop(0, NCH, body, 0)
    plsc.subcore_barrier()

    pltpu.sync_copy(acc_sh.at[pl.ds(s * RPT, RPT)],
                    out_acc.at[c, pl.ds(s * RPT, RPT)])
    if with_deg:
        pltpu.sync_copy(deg_sh.at[pl.ds(s * RPT, RPT)],
                        out_deg.at[c, pl.ds(s * RPT, RPT)])


def _make_sc_scatter(d, with_deg):
    mesh = plsc.VectorSubcoreMesh(core_axis_name="c", subcore_axis_name="s",
                                  num_cores=NC, num_subcores=NS)
    out_type = [jax.ShapeDtypeStruct((NC, NP, d), jnp.float32)]
    if with_deg:
        out_type.append(jax.ShapeDtypeStruct((NC, NP), jnp.float32))
    scratch = [
        pltpu.VMEM((NCH, CHUNK), jnp.int32),
        pltpu.VMEM((NCH, CHUNK), jnp.int32),
        pltpu.VMEM((CHUNK, d), jnp.float32),
        pltpu.VMEM((CHUNK,), jnp.float32),
        pltpu.VMEM_SHARED((NP, d), jnp.float32),
        pltpu.VMEM_SHARED((NP,), jnp.float32),
        pltpu.SemaphoreType.DMA,
    ]

    if with_deg:
        def body(table, srcs, dsts, zrows, zdeg, ones_in, out_acc, out_deg,
                 src_v, dst_v, rows_v, ones_v, acc_sh, deg_sh, sem):
            _sc_body(True, d, table, srcs, dsts, zrows, zdeg, ones_in,
                     out_acc, out_deg, src_v, dst_v, rows_v, ones_v, acc_sh,
                     deg_sh, sem)
    else:
        def body(table, srcs, dsts, zrows, out_acc,
                 src_v, dst_v, rows_v, ones_v, acc_sh, deg_sh, sem):
            _sc_body(False, d, table, srcs, dsts, zrows, None, None,
                     out_acc, None, src_v, dst_v, rows_v, ones_v, acc_sh,
                     deg_sh, sem)

    return pl.kernel(body, out_type=out_type, mesh=mesh,
                     scratch_types=scratch)


BM = 1000


def _pre_body(x_ref, wl_ref, wr_ref, bl_ref, a_ref, r_ref):
    xb = x_ref[...]
    a_ref[...] = jnp.dot(xb, wl_ref[...], preferred_element_type=jnp.float32)
    r_ref[...] = (jnp.dot(xb, wr_ref[...], preferred_element_type=jnp.float32)
                  + bl_ref[...])


def _mid_body(sa_ref, sb_ref, da_ref, db_ref, r_ref, wl_ref, wr_ref, bl_ref,
              a_ref, rn_ref):
    invd = 1.0 / jnp.maximum(da_ref[...] + db_ref[...], 1.0)
    h = jnp.maximum((sa_ref[...] + sb_ref[...]) * invd + r_ref[...], 0.0)
    a_ref[...] = jnp.dot(h, wl_ref[...], preferred_element_type=jnp.float32)
    rn_ref[...] = (jnp.dot(h, wr_ref[...], preferred_element_type=jnp.float32)
                   + bl_ref[...])


def _mid2_body(sa_ref, sb_ref, da_ref, db_ref, r_ref, wr_ref, bl_ref,
               h_ref, rn_ref):
    invd = 1.0 / jnp.maximum(da_ref[...] + db_ref[...], 1.0)
    h = jnp.maximum((sa_ref[...] + sb_ref[...]) * invd + r_ref[...], 0.0)
    h_ref[...] = h
    rn_ref[...] = (jnp.dot(h, wr_ref[...], preferred_element_type=jnp.float32)
                   + bl_ref[...])


def _fin_body(sa_ref, sb_ref, da_ref, db_ref, r_ref, wl_ref, o_ref):
    invd = 1.0 / jnp.maximum(da_ref[...] + db_ref[...], 1.0)
    mean = (sa_ref[...] + sb_ref[...]) * invd
    o_ref[...] = (jnp.dot(mean, wl_ref[...], preferred_element_type=jnp.float32)
                  + r_ref[...])


def _row_spec(dcol):
    return pl.BlockSpec((BM, dcol), lambda i: (i, 0))


def _full_spec(r, c):
    return pl.BlockSpec((r, c), lambda i: (0, 0))


def _tc_pre(x, wlT, wrT, bl, dout):
    din = x.shape[1]
    return pl.pallas_call(
        _pre_body,
        grid=(N // BM,),
        in_specs=[_row_spec(din), _full_spec(din, dout), _full_spec(din, dout),
                  _full_spec(1, dout)],
        out_specs=[_row_spec(dout), _row_spec(dout)],
        out_shape=[jax.ShapeDtypeStruct((N, dout), jnp.float32)] * 2,
    )(x, wlT, wrT, bl)


def _tc_mid(sa, sb, da, db, r, wlT, wrT, bl, dout):
    din = sa.shape[1]
    return pl.pallas_call(
        _mid_body,
        grid=(N // BM,),
        in_specs=[_row_spec(din), _row_spec(din), _row_spec(1), _row_spec(1),
                  _row_spec(din), _full_spec(din, dout), _full_spec(din, dout),
                  _full_spec(1, dout)],
        out_specs=[_row_spec(dout), _row_spec(dout)],
        out_shape=[jax.ShapeDtypeStruct((N, dout), jnp.float32)] * 2,
    )(sa, sb, da, db, r, wlT, wrT, bl)


def _tc_mid2(sa, sb, da, db, r, wrT, bl, dout):
    din = sa.shape[1]
    return pl.pallas_call(
        _mid2_body,
        grid=(N // BM,),
        in_specs=[_row_spec(din), _row_spec(din), _row_spec(1), _row_spec(1),
                  _row_spec(din), _full_spec(din, dout), _full_spec(1, dout)],
        out_specs=[_row_spec(din), _row_spec(dout)],
        out_shape=[jax.ShapeDtypeStruct((N, din), jnp.float32),
                   jax.ShapeDtypeStruct((N, dout), jnp.float32)],
    )(sa, sb, da, db, r, wrT, bl)


def _tc_fin(sa, sb, da, db, r, wlT, dout):
    din = sa.shape[1]
    return pl.pallas_call(
        _fin_body,
        grid=(N // BM,),
        in_specs=[_row_spec(din), _row_spec(din), _row_spec(1), _row_spec(1),
                  _row_spec(dout), _full_spec(din, dout)],
        out_specs=_row_spec(dout),
        out_shape=jax.ShapeDtypeStruct((N, dout), jnp.float32),
    )(sa, sb, da, db, r, wlT)


@jax.jit
def kernel(x, edge_index, Wl0, bl0, Wr0, Wl1, bl1, Wr1, Wl2, bl2, Wr2):
    src = edge_index[0]
    dst = edge_index[1]
    pad = EPAD - E
    srcs = jnp.concatenate([src, jnp.zeros((pad,), jnp.int32)])
    dsts = jnp.concatenate([dst, jnp.full((pad,), NP - 1, jnp.int32)])
    srcs = srcs.reshape(NW, NCH, CHUNK)
    dsts = dsts.reshape(NW, NCH, CHUNK)

    zrows128 = jnp.zeros((RPT, D_HID), jnp.float32)
    zdeg = jnp.zeros((RPT,), jnp.float32)
    ones_in = jnp.ones((CHUNK,), jnp.float32)

    sc0 = _make_sc_scatter(D_HID, True)
    sc1 = _make_sc_scatter(D_HID, False)

    a0, r0 = _tc_pre(x, Wl0.T, Wr0.T, bl0[None, :], D_HID)
    s0, deg = sc0(a0, srcs, dsts, zrows128, zdeg, ones_in)
    da = deg[0, :N, None]
    db = deg[1, :N, None]

    a1, r1 = _tc_mid(s0[0, :N], s0[1, :N], da, db, r0,
                     Wl1.T, Wr1.T, bl1[None, :], D_HID)
    (s1,) = sc1(a1, srcs, dsts, zrows128)

    h2, r2 = _tc_mid2(s1[0, :N], s1[1, :N], da, db, r1,
                      Wr2.T, bl2[None, :], D_OUT)
    (s2,) = sc1(h2, srcs, dsts, zrows128)

    return _tc_fin(s2[0, :N], s2[1, :N], da, db, r2, Wl2.T, D_OUT)


# speedup vs baseline: 7.4511x; 2.2251x over previous
import functools

import jax
import jax.numpy as jnp
from jax import lax
from jax.experimental import pallas as pl
from jax.experimental.pallas import tpu as pltpu
from jax.experimental.pallas import tpu_sc as plsc

N = 10000
E = 160000
D_IN = 256
D_HID = 128
D_OUT = 64

NC = 2
NS = 16
NW = NC * NS
CHUNK = 128
NCH = (E + NW * CHUNK - 1) // (NW * CHUNK)
EPAD = NW * NCH * CHUNK
NP = 10240
RPT = NP // NS


def _sc_body(with_deg, d, table, srcs, dsts, zrows, zdeg, ones_in,
             out_acc, out_deg, src_v, dst_v, rows_a, rows_b, ones_v, acc_sh,
             deg_sh, sem):
    c = lax.axis_index("c")
    s = lax.axis_index("s")
    wid = s * NC + c
    trip2 = jnp.where(wid == NW - 1, (E - (NW - 1) * NCH * CHUNK) // CHUNK,
                      NCH) // 2

    pltpu.sync_copy(zrows, acc_sh.at[pl.ds(s * RPT, RPT)])
    pltpu.sync_copy(srcs.at[wid], src_v)
    pltpu.sync_copy(dsts.at[wid], dst_v)
    if with_deg:
        pltpu.sync_copy(zdeg, deg_sh.at[pl.ds(s * RPT, RPT)])
        pltpu.sync_copy(ones_in, ones_v)
    plsc.subcore_barrier()

    def body(j, carry):
        pltpu.async_copy(table.at[src_v.at[j]], rows_a, sem).wait()
        pltpu.sync_copy(rows_a, acc_sh.at[dst_v.at[j]], add=True)
        if with_deg:
            pltpu.sync_copy(ones_v, deg_sh.at[dst_v.at[j]], add=True)
        return carry

    lax.fori_loop(0, trip2 * 2, body, 0)
    plsc.subcore_barrier()

    pltpu.sync_copy(acc_sh.at[pl.ds(s * RPT, RPT)],
                    out_acc.at[c, pl.ds(s * RPT, RPT)])
    if with_deg:
        pltpu.sync_copy(deg_sh.at[pl.ds(s * RPT, RPT)],
                        out_deg.at[c, pl.ds(s * RPT, RPT)])


def _make_sc_scatter(d, with_deg):
    mesh = plsc.VectorSubcoreMesh(core_axis_name="c", subcore_axis_name="s",
                                  num_cores=NC, num_subcores=NS)
    out_type = [jax.ShapeDtypeStruct((NC, NP, d), jnp.float32)]
    if with_deg:
        out_type.append(jax.ShapeDtypeStruct((NC, NP), jnp.float32))
    scratch = [
        pltpu.VMEM((NCH, CHUNK), jnp.int32),
        pltpu.VMEM((NCH, CHUNK), jnp.int32),
        pltpu.VMEM((CHUNK, d), jnp.float32),
        pltpu.VMEM((CHUNK, d), jnp.float32),
        pltpu.VMEM((CHUNK,), jnp.float32),
        pltpu.VMEM_SHARED((NP, d), jnp.float32),
        pltpu.VMEM_SHARED((NP,), jnp.float32),
        pltpu.SemaphoreType.DMA,
    ]

    if with_deg:
        def body(table, srcs, dsts, zrows, zdeg, ones_in, out_acc, out_deg,
                 src_v, dst_v, rows_a, rows_b, ones_v, acc_sh, deg_sh, sem):
            _sc_body(True, d, table, srcs, dsts, zrows, zdeg, ones_in,
                     out_acc, out_deg, src_v, dst_v, rows_a, rows_b, ones_v,
                     acc_sh, deg_sh, sem)
    else:
        def body(table, srcs, dsts, zrows, out_acc,
                 src_v, dst_v, rows_a, rows_b, ones_v, acc_sh, deg_sh, sem):
            _sc_body(False, d, table, srcs, dsts, zrows, None, None,
                     out_acc, None, src_v, dst_v, rows_a, rows_b, ones_v,
                     acc_sh, deg_sh, sem)

    return pl.kernel(body, out_type=out_type, mesh=mesh,
                     scratch_types=scratch)


BM = 1000


def _pre_body(x_ref, wl_ref, wr_ref, bl_ref, a_ref, r_ref):
    xb = x_ref[...]
    a_ref[...] = jnp.dot(xb, wl_ref[...], preferred_element_type=jnp.float32)
    r_ref[...] = (jnp.dot(xb, wr_ref[...], preferred_element_type=jnp.float32)
                  + bl_ref[...])


def _mid_body(sa_ref, sb_ref, da_ref, db_ref, r_ref, wl_ref, wr_ref, bl_ref,
              a_ref, rn_ref):
    invd = 1.0 / jnp.maximum(da_ref[...] + db_ref[...], 1.0)
    h = jnp.maximum((sa_ref[...] + sb_ref[...]) * invd + r_ref[...], 0.0)
    a_ref[...] = jnp.dot(h, wl_ref[...], preferred_element_type=jnp.float32)
    rn_ref[...] = (jnp.dot(h, wr_ref[...], preferred_element_type=jnp.float32)
                   + bl_ref[...])


def _mid2_body(sa_ref, sb_ref, da_ref, db_ref, r_ref, wr_ref, bl_ref,
               h_ref, rn_ref):
    invd = 1.0 / jnp.maximum(da_ref[...] + db_ref[...], 1.0)
    h = jnp.maximum((sa_ref[...] + sb_ref[...]) * invd + r_ref[...], 0.0)
    h_ref[...] = h
    rn_ref[...] = (jnp.dot(h, wr_ref[...], preferred_element_type=jnp.float32)
                   + bl_ref[...])


def _fin_body(sa_ref, sb_ref, da_ref, db_ref, r_ref, wl_ref, o_ref):
    invd = 1.0 / jnp.maximum(da_ref[...] + db_ref[...], 1.0)
    mean = (sa_ref[...] + sb_ref[...]) * invd
    o_ref[...] = (jnp.dot(mean, wl_ref[...], preferred_element_type=jnp.float32)
                  + r_ref[...])


def _row_spec(dcol):
    return pl.BlockSpec((BM, dcol), lambda i: (i, 0))


def _full_spec(r, c):
    return pl.BlockSpec((r, c), lambda i: (0, 0))


def _tc_pre(x, wlT, wrT, bl, dout):
    din = x.shape[1]
    return pl.pallas_call(
        _pre_body,
        grid=(N // BM,),
        in_specs=[_row_spec(din), _full_spec(din, dout), _full_spec(din, dout),
                  _full_spec(1, dout)],
        out_specs=[_row_spec(dout), _row_spec(dout)],
        out_shape=[jax.ShapeDtypeStruct((N, dout), jnp.float32)] * 2,
    )(x, wlT, wrT, bl)


def _tc_mid(sa, sb, da, db, r, wlT, wrT, bl, dout):
    din = sa.shape[1]
    return pl.pallas_call(
        _mid_body,
        grid=(N // BM,),
        in_specs=[_row_spec(din), _row_spec(din), _row_spec(1), _row_spec(1),
                  _row_spec(din), _full_spec(din, dout), _full_spec(din, dout),
                  _full_spec(1, dout)],
        out_specs=[_row_spec(dout), _row_spec(dout)],
        out_shape=[jax.ShapeDtypeStruct((N, dout), jnp.float32)] * 2,
    )(sa, sb, da, db, r, wlT, wrT, bl)


def _tc_mid2(sa, sb, da, db, r, wrT, bl, dout):
    din = sa.shape[1]
    return pl.pallas_call(
        _mid2_body,
        grid=(N // BM,),
        in_specs=[_row_spec(din), _row_spec(din), _row_spec(1), _row_spec(1),
                  _row_spec(din), _full_spec(din, dout), _full_spec(1, dout)],
        out_specs=[_row_spec(din), _row_spec(dout)],
        out_shape=[jax.ShapeDtypeStruct((N, din), jnp.float32),
                   jax.ShapeDtypeStruct((N, dout), jnp.float32)],
    )(sa, sb, da, db, r, wrT, bl)


def _tc_fin(sa, sb, da, db, r, wlT, dout):
    din = sa.shape[1]
    return pl.pallas_call(
        _fin_body,
        grid=(N // BM,),
        in_specs=[_row_spec(din), _row_spec(din), _row_spec(1), _row_spec(1),
                  _row_spec(dout), _full_spec(din, dout)],
        out_specs=_row_spec(dout),
        out_shape=jax.ShapeDtypeStruct((N, dout), jnp.float32),
    )(sa, sb, da, db, r, wlT)


@jax.jit
def kernel(x, edge_index, Wl0, bl0, Wr0, Wl1, bl1, Wr1, Wl2, bl2, Wr2):
    src = edge_index[0]
    dst = edge_index[1]
    pad = EPAD - E
    srcs = jnp.concatenate([src, jnp.zeros((pad,), jnp.int32)])
    dsts = jnp.concatenate([dst, jnp.full((pad,), NP - 1, jnp.int32)])
    srcs = srcs.reshape(NW, NCH, CHUNK)
    dsts = dsts.reshape(NW, NCH, CHUNK)

    zrows128 = jnp.zeros((RPT, D_HID), jnp.float32)
    zdeg = jnp.zeros((RPT,), jnp.float32)
    ones_in = jnp.ones((CHUNK,), jnp.float32)

    sc0 = _make_sc_scatter(D_HID, True)
    sc1 = _make_sc_scatter(D_HID, False)

    a0, r0 = _tc_pre(x, Wl0.T, Wr0.T, bl0[None, :], D_HID)
    s0, deg = sc0(a0, srcs, dsts, zrows128, zdeg, ones_in)
    da = deg[0, :N, None]
    db = deg[1, :N, None]

    a1, r1 = _tc_mid(s0[0, :N], s0[1, :N], da, db, r0,
                     Wl1.T, Wr1.T, bl1[None, :], D_HID)
    (s1,) = sc1(a1, srcs, dsts, zrows128)

    h2, r2 = _tc_mid2(s1[0, :N], s1[1, :N], da, db, r1,
                      Wr2.T, bl2[None, :], D_OUT)
    (s2,) = sc1(h2, srcs, dsts, zrows128)

    return _tc_fin(s2[0, :N], s2[1, :N], da, db, r2, Wl2.T, D_OUT)


# speedup vs baseline: 8.1715x; 1.0967x over previous
import functools

import jax
import jax.numpy as jnp
from jax import lax
from jax.experimental import pallas as pl
from jax.experimental.pallas import tpu as pltpu
from jax.experimental.pallas import tpu_sc as plsc

N = 10000
E = 160000
D_IN = 256
D_HID = 128
D_OUT = 64

NC = 2
NS = 16
NW = NC * NS
CHUNK = 128
NCH = (E + NW * CHUNK - 1) // (NW * CHUNK)
EPAD = NW * NCH * CHUNK
NP = 10240
RPT = NP // NS


def _sc_body(with_deg, d, table, srcs, dsts, zrows, zdeg, ones_in,
             out_acc, out_deg, src_v, dst_v, rows_a, rows_b, ones_v, acc_sh,
             deg_sh, sem, semb):
    c = lax.axis_index("c")
    s = lax.axis_index("s")
    wid = s * NC + c
    trip2 = jnp.where(wid == NW - 1, (E - (NW - 1) * NCH * CHUNK) // CHUNK,
                      NCH) // 2

    pltpu.sync_copy(zrows, acc_sh.at[pl.ds(s * RPT, RPT)])
    pltpu.sync_copy(srcs.at[wid], src_v)
    pltpu.sync_copy(dsts.at[wid], dst_v)
    if with_deg:
        pltpu.sync_copy(zdeg, deg_sh.at[pl.ds(s * RPT, RPT)])
        pltpu.sync_copy(ones_in, ones_v)
    plsc.subcore_barrier()

    def body(j2, carry):
        j = 2 * j2
        ha = pltpu.async_copy(table.at[src_v.at[j]], rows_a, sem)
        hb = pltpu.async_copy(table.at[src_v.at[j + 1]], rows_b, semb)
        ha.wait()
        pltpu.sync_copy(rows_a, acc_sh.at[dst_v.at[j]], add=True)
        if with_deg:
            pltpu.sync_copy(ones_v, deg_sh.at[dst_v.at[j]], add=True)
        hb.wait()
        pltpu.sync_copy(rows_b, acc_sh.at[dst_v.at[j + 1]], add=True)
        if with_deg:
            pltpu.sync_copy(ones_v, deg_sh.at[dst_v.at[j + 1]], add=True)
        return carry

    lax.fori_loop(0, trip2, body, 0)
    plsc.subcore_barrier()

    pltpu.sync_copy(acc_sh.at[pl.ds(s * RPT, RPT)],
                    out_acc.at[c, pl.ds(s * RPT, RPT)])
    if with_deg:
        pltpu.sync_copy(deg_sh.at[pl.ds(s * RPT, RPT)],
                        out_deg.at[c, pl.ds(s * RPT, RPT)])


def _make_sc_scatter(d, with_deg):
    mesh = plsc.VectorSubcoreMesh(core_axis_name="c", subcore_axis_name="s",
                                  num_cores=NC, num_subcores=NS)
    out_type = [jax.ShapeDtypeStruct((NC, NP, d), jnp.float32)]
    if with_deg:
        out_type.append(jax.ShapeDtypeStruct((NC, NP), jnp.float32))
    scratch = [
        pltpu.VMEM((NCH, CHUNK), jnp.int32),
        pltpu.VMEM((NCH, CHUNK), jnp.int32),
        pltpu.VMEM((CHUNK, d), jnp.float32),
        pltpu.VMEM((CHUNK, d), jnp.float32),
        pltpu.VMEM((CHUNK,), jnp.float32),
        pltpu.VMEM_SHARED((NP, d), jnp.float32),
        pltpu.VMEM_SHARED((NP,), jnp.float32),
        pltpu.SemaphoreType.DMA,
        pltpu.SemaphoreType.DMA,
    ]

    if with_deg:
        def body(table, srcs, dsts, zrows, zdeg, ones_in, out_acc, out_deg,
                 src_v, dst_v, rows_a, rows_b, ones_v, acc_sh, deg_sh,
                 sem, semb):
            _sc_body(True, d, table, srcs, dsts, zrows, zdeg, ones_in,
                     out_acc, out_deg, src_v, dst_v, rows_a, rows_b, ones_v,
                     acc_sh, deg_sh, sem, semb)
    else:
        def body(table, srcs, dsts, zrows, out_acc,
                 src_v, dst_v, rows_a, rows_b, ones_v, acc_sh, deg_sh,
                 sem, semb):
            _sc_body(False, d, table, srcs, dsts, zrows, None, None,
                     out_acc, None, src_v, dst_v, rows_a, rows_b, ones_v,
                     acc_sh, deg_sh, sem, semb)

    return pl.kernel(body, out_type=out_type, mesh=mesh,
                     scratch_types=scratch)


BM = 1000


def _pre_body(x_ref, wl_ref, wr_ref, bl_ref, a_ref, r_ref):
    xb = x_ref[...]
    a_ref[...] = jnp.dot(xb, wl_ref[...], preferred_element_type=jnp.float32)
    r_ref[...] = (jnp.dot(xb, wr_ref[...], preferred_element_type=jnp.float32)
                  + bl_ref[...])


def _mid_body(sa_ref, sb_ref, da_ref, db_ref, r_ref, wl_ref, wr_ref, bl_ref,
              a_ref, rn_ref):
    invd = 1.0 / jnp.maximum(da_ref[...] + db_ref[...], 1.0)
    h = jnp.maximum((sa_ref[...] + sb_ref[...]) * invd + r_ref[...], 0.0)
    a_ref[...] = jnp.dot(h, wl_ref[...], preferred_element_type=jnp.float32)
    rn_ref[...] = (jnp.dot(h, wr_ref[...], preferred_element_type=jnp.float32)
                   + bl_ref[...])


def _mid2_body(sa_ref, sb_ref, da_ref, db_ref, r_ref, wr_ref, bl_ref,
               h_ref, rn_ref):
    invd = 1.0 / jnp.maximum(da_ref[...] + db_ref[...], 1.0)
    h = jnp.maximum((sa_ref[...] + sb_ref[...]) * invd + r_ref[...], 0.0)
    h_ref[...] = h
    rn_ref[...] = (jnp.dot(h, wr_ref[...], preferred_element_type=jnp.float32)
                   + bl_ref[...])


def _fin_body(sa_ref, sb_ref, da_ref, db_ref, r_ref, wl_ref, o_ref):
    invd = 1.0 / jnp.maximum(da_ref[...] + db_ref[...], 1.0)
    mean = (sa_ref[...] + sb_ref[...]) * invd
    o_ref[...] = (jnp.dot(mean, wl_ref[...], preferred_element_type=jnp.float32)
                  + r_ref[...])


def _row_spec(dcol):
    return pl.BlockSpec((BM, dcol), lambda i: (i, 0))


def _full_spec(r, c):
    return pl.BlockSpec((r, c), lambda i: (0, 0))


def _tc_pre(x, wlT, wrT, bl, dout):
    din = x.shape[1]
    return pl.pallas_call(
        _pre_body,
        grid=(N // BM,),
        in_specs=[_row_spec(din), _full_spec(din, dout), _full_spec(din, dout),
                  _full_spec(1, dout)],
        out_specs=[_row_spec(dout), _row_spec(dout)],
        out_shape=[jax.ShapeDtypeStruct((N, dout), jnp.float32)] * 2,
    )(x, wlT, wrT, bl)


def _tc_mid(sa, sb, da, db, r, wlT, wrT, bl, dout):
    din = sa.shape[1]
    return pl.pallas_call(
        _mid_body,
        grid=(N // BM,),
        in_specs=[_row_spec(din), _row_spec(din), _row_spec(1), _row_spec(1),
                  _row_spec(din), _full_spec(din, dout), _full_spec(din, dout),
                  _full_spec(1, dout)],
        out_specs=[_row_spec(dout), _row_spec(dout)],
        out_shape=[jax.ShapeDtypeStruct((N, dout), jnp.float32)] * 2,
    )(sa, sb, da, db, r, wlT, wrT, bl)


def _tc_mid2(sa, sb, da, db, r, wrT, bl, dout):
    din = sa.shape[1]
    return pl.pallas_call(
        _mid2_body,
        grid=(N // BM,),
        in_specs=[_row_spec(din), _row_spec(din), _row_spec(1), _row_spec(1),
                  _row_spec(din), _full_spec(din, dout), _full_spec(1, dout)],
        out_specs=[_row_spec(din), _row_spec(dout)],
        out_shape=[jax.ShapeDtypeStruct((N, din), jnp.float32),
                   jax.ShapeDtypeStruct((N, dout), jnp.float32)],
    )(sa, sb, da, db, r, wrT, bl)


def _tc_fin(sa, sb, da, db, r, wlT, dout):
    din = sa.shape[1]
    return pl.pallas_call(
        _fin_body,
        grid=(N // BM,),
        in_specs=[_row_spec(din), _row_spec(din), _row_spec(1), _row_spec(1),
                  _row_spec(dout), _full_spec(din, dout)],
        out_specs=_row_spec(dout),
        out_shape=jax.ShapeDtypeStruct((N, dout), jnp.float32),
    )(sa, sb, da, db, r, wlT)


@jax.jit
def kernel(x, edge_index, Wl0, bl0, Wr0, Wl1, bl1, Wr1, Wl2, bl2, Wr2):
    src = edge_index[0]
    dst = edge_index[1]
    pad = EPAD - E
    srcs = jnp.concatenate([src, jnp.zeros((pad,), jnp.int32)])
    dsts = jnp.concatenate([dst, jnp.full((pad,), NP - 1, jnp.int32)])
    srcs = srcs.reshape(NW, NCH, CHUNK)
    dsts = dsts.reshape(NW, NCH, CHUNK)

    zrows128 = jnp.zeros((RPT, D_HID), jnp.float32)
    zdeg = jnp.zeros((RPT,), jnp.float32)
    ones_in = jnp.ones((CHUNK,), jnp.float32)

    sc0 = _make_sc_scatter(D_HID, True)
    sc1 = _make_sc_scatter(D_HID, False)

    a0, r0 = _tc_pre(x, Wl0.T, Wr0.T, bl0[None, :], D_HID)
    s0, deg = sc0(a0, srcs, dsts, zrows128, zdeg, ones_in)
    da = deg[0, :N, None]
    db = deg[1, :N, None]

    a1, r1 = _tc_mid(s0[0, :N], s0[1, :N], da, db, r0,
                     Wl1.T, Wr1.T, bl1[None, :], D_HID)
    (s1,) = sc1(a1, srcs, dsts, zrows128)

    h2, r2 = _tc_mid2(s1[0, :N], s1[1, :N], da, db, r1,
                      Wr2.T, bl2[None, :], D_OUT)
    (s2,) = sc1(h2, srcs, dsts, zrows128)

    return _tc_fin(s2[0, :N], s2[1, :N], da, db, r2, Wl2.T, D_OUT)


# speedup vs baseline: 8.6985x; 1.0645x over previous
import functools

import jax
import jax.numpy as jnp
from jax import lax
from jax.experimental import pallas as pl
from jax.experimental.pallas import tpu as pltpu
from jax.experimental.pallas import tpu_sc as plsc

N = 10000
E = 160000
D_IN = 256
D_HID = 128
D_OUT = 64

NC = 2
NS = 16
NW = NC * NS
CHUNK = 128
NCH = (E + NW * CHUNK - 1) // (NW * CHUNK)
EPAD = NW * NCH * CHUNK
NP = 10240
RPT = NP // NS


def _sc_body(with_deg, d, table, srcs, dsts, zrows, zdeg, ones_in,
             out_acc, out_deg, src_v, dst_v, rows_a, rows_b, ones_v, acc_sh,
             deg_sh, sem, semb):
    c = lax.axis_index("c")
    s = lax.axis_index("s")
    wid = s * NC + c
    trip2 = jnp.where(wid == NW - 1, (E - (NW - 1) * NCH * CHUNK) // CHUNK,
                      NCH) // 2

    pltpu.sync_copy(zrows, acc_sh.at[pl.ds(s * RPT, RPT)])
    pltpu.sync_copy(srcs.at[wid], src_v)
    pltpu.sync_copy(dsts.at[wid], dst_v)
    if with_deg:
        pltpu.sync_copy(zdeg, deg_sh.at[pl.ds(s * RPT, RPT)])
        pltpu.sync_copy(ones_in, ones_v)
    plsc.subcore_barrier()

    def body(j2, carry):
        j = 2 * j2
        ha = pltpu.async_copy(table.at[src_v.at[j]], rows_a, sem)
        hb = pltpu.async_copy(table.at[src_v.at[j + 1]], rows_b, semb)
        ha.wait()
        pltpu.sync_copy(rows_a, acc_sh.at[dst_v.at[j]], add=True)
        if with_deg:
            pltpu.sync_copy(ones_v, deg_sh.at[dst_v.at[j]], add=True)
        hb.wait()
        pltpu.sync_copy(rows_b, acc_sh.at[dst_v.at[j + 1]], add=True)
        if with_deg:
            pltpu.sync_copy(ones_v, deg_sh.at[dst_v.at[j + 1]], add=True)
        return carry

    lax.fori_loop(0, trip2, body, 0)
    plsc.subcore_barrier()

    pltpu.sync_copy(acc_sh.at[pl.ds(s * RPT, RPT)],
                    out_acc.at[c, pl.ds(s * RPT, RPT)])
    if with_deg:
        pltpu.sync_copy(deg_sh.at[pl.ds(s * RPT, RPT)],
                        out_deg.at[c, pl.ds(s * RPT, RPT)])


def _make_sc_scatter(d, with_deg):
    mesh = plsc.VectorSubcoreMesh(core_axis_name="c", subcore_axis_name="s",
                                  num_cores=NC, num_subcores=NS)
    out_type = [jax.ShapeDtypeStruct((NC, NP, d), jnp.float32)]
    if with_deg:
        out_type.append(jax.ShapeDtypeStruct((NC, NP), jnp.float32))
    scratch = [
        pltpu.VMEM((NCH, CHUNK), jnp.int32),
        pltpu.VMEM((NCH, CHUNK), jnp.int32),
        pltpu.VMEM((CHUNK, d), jnp.float32),
        pltpu.VMEM((CHUNK, d), jnp.float32),
        pltpu.VMEM((CHUNK,), jnp.float32),
        pltpu.VMEM_SHARED((NP, d), jnp.float32),
        pltpu.VMEM_SHARED((NP,), jnp.float32),
        pltpu.SemaphoreType.DMA,
        pltpu.SemaphoreType.DMA,
    ]

    if with_deg:
        def body(table, srcs, dsts, zrows, zdeg, ones_in, out_acc, out_deg,
                 src_v, dst_v, rows_a, rows_b, ones_v, acc_sh, deg_sh,
                 sem, semb):
            _sc_body(True, d, table, srcs, dsts, zrows, zdeg, ones_in,
                     out_acc, out_deg, src_v, dst_v, rows_a, rows_b, ones_v,
                     acc_sh, deg_sh, sem, semb)
    else:
        def body(table, srcs, dsts, zrows, out_acc,
                 src_v, dst_v, rows_a, rows_b, ones_v, acc_sh, deg_sh,
                 sem, semb):
            _sc_body(False, d, table, srcs, dsts, zrows, None, None,
                     out_acc, None, src_v, dst_v, rows_a, rows_b, ones_v,
                     acc_sh, deg_sh, sem, semb)

    return pl.kernel(body, out_type=out_type, mesh=mesh,
                     scratch_types=scratch)


BM = 1000


def _pre_body(x_ref, wl_ref, wr_ref, bl_ref, a_ref, r_ref):
    xb = x_ref[...]
    a_ref[...] = jnp.dot(xb, wl_ref[...], preferred_element_type=jnp.float32)
    r_ref[...] = (jnp.dot(xb, wr_ref[...], preferred_element_type=jnp.float32)
                  + bl_ref[...])


def _mid_body(sa_ref, sb_ref, da_ref, db_ref, r_ref, wl_ref, wr_ref, bl_ref,
              a_ref, rn_ref):
    invd = 1.0 / jnp.maximum(da_ref[...] + db_ref[...], 1.0)
    h = jnp.maximum((sa_ref[0] + sb_ref[0]) * invd + r_ref[...], 0.0)
    a_ref[...] = jnp.dot(h, wl_ref[...], preferred_element_type=jnp.float32)
    rn_ref[...] = (jnp.dot(h, wr_ref[...], preferred_element_type=jnp.float32)
                   + bl_ref[...])


def _mid2_body(sa_ref, sb_ref, da_ref, db_ref, r_ref, wr_ref, bl_ref,
               h_ref, rn_ref):
    invd = 1.0 / jnp.maximum(da_ref[...] + db_ref[...], 1.0)
    h = jnp.maximum((sa_ref[0] + sb_ref[0]) * invd + r_ref[...], 0.0)
    h_ref[...] = h
    rn_ref[...] = (jnp.dot(h, wr_ref[...], preferred_element_type=jnp.float32)
                   + bl_ref[...])


def _fin_body(sa_ref, sb_ref, da_ref, db_ref, r_ref, wl_ref, o_ref):
    invd = 1.0 / jnp.maximum(da_ref[...] + db_ref[...], 1.0)
    mean = (sa_ref[0] + sb_ref[0]) * invd
    o_ref[...] = (jnp.dot(mean, wl_ref[...], preferred_element_type=jnp.float32)
                  + r_ref[...])


def _row_spec(dcol):
    return pl.BlockSpec((BM, dcol), lambda i: (i, 0))


def _part_spec(dcol, core):
    return pl.BlockSpec((1, BM, dcol), lambda i, c=core: (c, i, 0))


def _full_spec(r, c):
    return pl.BlockSpec((r, c), lambda i: (0, 0))


def _tc_pre(x, wlT, wrT, bl, dout):
    din = x.shape[1]
    return pl.pallas_call(
        _pre_body,
        grid=(N // BM,),
        in_specs=[_row_spec(din), _full_spec(din, dout), _full_spec(din, dout),
                  _full_spec(1, dout)],
        out_specs=[_row_spec(dout), _row_spec(dout)],
        out_shape=[jax.ShapeDtypeStruct((N, dout), jnp.float32)] * 2,
    )(x, wlT, wrT, bl)


def _tc_mid(s2c, da, db, r, wlT, wrT, bl, dout):
    din = s2c.shape[2]
    sa = sb = s2c
    return pl.pallas_call(
        _mid_body,
        grid=(N // BM,),
        in_specs=[_part_spec(din, 0), _part_spec(din, 1), _row_spec(1),
                  _row_spec(1), _row_spec(din), _full_spec(din, dout),
                  _full_spec(din, dout), _full_spec(1, dout)],
        out_specs=[_row_spec(dout), _row_spec(dout)],
        out_shape=[jax.ShapeDtypeStruct((N, dout), jnp.float32)] * 2,
    )(sa, sb, da, db, r, wlT, wrT, bl)


def _tc_mid2(s2c, da, db, r, wrT, bl, dout):
    din = s2c.shape[2]
    sa = sb = s2c
    return pl.pallas_call(
        _mid2_body,
        grid=(N // BM,),
        in_specs=[_part_spec(din, 0), _part_spec(din, 1), _row_spec(1),
                  _row_spec(1), _row_spec(din), _full_spec(din, dout),
                  _full_spec(1, dout)],
        out_specs=[_row_spec(din), _row_spec(dout)],
        out_shape=[jax.ShapeDtypeStruct((N, din), jnp.float32),
                   jax.ShapeDtypeStruct((N, dout), jnp.float32)],
    )(sa, sb, da, db, r, wrT, bl)


def _tc_fin(s2c, da, db, r, wlT, dout):
    din = s2c.shape[2]
    sa = sb = s2c
    return pl.pallas_call(
        _fin_body,
        grid=(N // BM,),
        in_specs=[_part_spec(din, 0), _part_spec(din, 1), _row_spec(1),
                  _row_spec(1), _row_spec(dout), _full_spec(din, dout)],
        out_specs=_row_spec(dout),
        out_shape=jax.ShapeDtypeStruct((N, dout), jnp.float32),
    )(sa, sb, da, db, r, wlT)


@jax.jit
def kernel(x, edge_index, Wl0, bl0, Wr0, Wl1, bl1, Wr1, Wl2, bl2, Wr2):
    src = edge_index[0]
    dst = edge_index[1]
    pad = EPAD - E
    srcs = jnp.concatenate([src, jnp.zeros((pad,), jnp.int32)])
    dsts = jnp.concatenate([dst, jnp.full((pad,), NP - 1, jnp.int32)])
    srcs = srcs.reshape(NW, NCH, CHUNK)
    dsts = dsts.reshape(NW, NCH, CHUNK)

    zrows128 = jnp.zeros((RPT, D_HID), jnp.float32)
    zdeg = jnp.zeros((RPT,), jnp.float32)
    ones_in = jnp.ones((CHUNK,), jnp.float32)

    sc0 = _make_sc_scatter(D_HID, True)
    sc1 = _make_sc_scatter(D_HID, False)

    a0, r0 = _tc_pre(x, Wl0.T, Wr0.T, bl0[None, :], D_HID)
    s0, deg = sc0(a0, srcs, dsts, zrows128, zdeg, ones_in)
    da = deg[0, :N, None]
    db = deg[1, :N, None]

    a1, r1 = _tc_mid(s0, da, db, r0, Wl1.T, Wr1.T, bl1[None, :], D_HID)
    (s1,) = sc1(a1, srcs, dsts, zrows128)

    h2, r2 = _tc_mid2(s1, da, db, r1, Wr2.T, bl2[None, :], D_OUT)
    (s2,) = sc1(h2, srcs, dsts, zrows128)

    return _tc_fin(s2, da, db, r2, Wl2.T, D_OUT)


# speedup vs baseline: 8.8272x; 1.0148x over previous
import functools

import jax
import jax.numpy as jnp
from jax import lax
from jax.experimental import pallas as pl
from jax.experimental.pallas import tpu as pltpu
from jax.experimental.pallas import tpu_sc as plsc

N = 10000
E = 160000
D_IN = 256
D_HID = 128
D_OUT = 64

NC = 2
NS = 16
NW = NC * NS
CHUNK = 128
NCH = (E + NW * CHUNK - 1) // (NW * CHUNK)
EPAD = NW * NCH * CHUNK
NP = 10240
RPT = NP // NS


def _sc_body(with_deg, d, table, srcs, dsts, zrows, zdeg, ones_in,
             out_acc, out_deg, src_v, dst_v, rows_a, rows_b, ones_v, acc_sh,
             deg_sh, sem, semb, semc, semd, seme, semf):
    c = lax.axis_index("c")
    s = lax.axis_index("s")
    wid = s * NC + c
    trip2 = jnp.where(wid == NW - 1, (E - (NW - 1) * NCH * CHUNK) // CHUNK,
                      NCH) // 2

    pltpu.sync_copy(zrows, acc_sh.at[pl.ds(s * RPT, RPT)])
    pltpu.sync_copy(srcs.at[wid], src_v)
    pltpu.sync_copy(dsts.at[wid], dst_v)
    if with_deg:
        pltpu.sync_copy(zdeg, deg_sh.at[pl.ds(s * RPT, RPT)])
        pltpu.sync_copy(ones_in, ones_v)
    plsc.subcore_barrier()

    def body(j2, carry):
        j = 2 * j2
        ha = pltpu.async_copy(table.at[src_v.at[j]], rows_a, sem)
        hb = pltpu.async_copy(table.at[src_v.at[j + 1]], rows_b, semb)
        ha.wait()
        sa = pltpu.async_copy(rows_a, acc_sh.at[dst_v.at[j]], semc, add=True)
        if with_deg:
            ea = pltpu.async_copy(ones_v, deg_sh.at[dst_v.at[j]], seme,
                                  add=True)
        hb.wait()
        sb = pltpu.async_copy(rows_b, acc_sh.at[dst_v.at[j + 1]], semd,
                              add=True)
        if with_deg:
            eb = pltpu.async_copy(ones_v, deg_sh.at[dst_v.at[j + 1]], semf,
                                  add=True)
        sa.wait()
        sb.wait()
        if with_deg:
            ea.wait()
            eb.wait()
        return carry

    lax.fori_loop(0, trip2, body, 0)
    plsc.subcore_barrier()

    pltpu.sync_copy(acc_sh.at[pl.ds(s * RPT, RPT)],
                    out_acc.at[c, pl.ds(s * RPT, RPT)])
    if with_deg:
        pltpu.sync_copy(deg_sh.at[pl.ds(s * RPT, RPT)],
                        out_deg.at[c, pl.ds(s * RPT, RPT)])


def _make_sc_scatter(d, with_deg):
    mesh = plsc.VectorSubcoreMesh(core_axis_name="c", subcore_axis_name="s",
                                  num_cores=NC, num_subcores=NS)
    out_type = [jax.ShapeDtypeStruct((NC, NP, d), jnp.float32)]
    if with_deg:
        out_type.append(jax.ShapeDtypeStruct((NC, NP), jnp.float32))
    scratch = [
        pltpu.VMEM((NCH, CHUNK), jnp.int32),
        pltpu.VMEM((NCH, CHUNK), jnp.int32),
        pltpu.VMEM((CHUNK, d), jnp.float32),
        pltpu.VMEM((CHUNK, d), jnp.float32),
        pltpu.VMEM((CHUNK,), jnp.float32),
        pltpu.VMEM_SHARED((NP, d), jnp.float32),
        pltpu.VMEM_SHARED((NP,), jnp.float32),
        pltpu.SemaphoreType.DMA,
        pltpu.SemaphoreType.DMA,
        pltpu.SemaphoreType.DMA,
        pltpu.SemaphoreType.DMA,
        pltpu.SemaphoreType.DMA,
        pltpu.SemaphoreType.DMA,
    ]

    if with_deg:
        def body(table, srcs, dsts, zrows, zdeg, ones_in, out_acc, out_deg,
                 src_v, dst_v, rows_a, rows_b, ones_v, acc_sh, deg_sh,
                 sem, semb, semc, semd, seme, semf):
            _sc_body(True, d, table, srcs, dsts, zrows, zdeg, ones_in,
                     out_acc, out_deg, src_v, dst_v, rows_a, rows_b, ones_v,
                     acc_sh, deg_sh, sem, semb, semc, semd, seme, semf)
    else:
        def body(table, srcs, dsts, zrows, out_acc,
                 src_v, dst_v, rows_a, rows_b, ones_v, acc_sh, deg_sh,
                 sem, semb, semc, semd, seme, semf):
            _sc_body(False, d, table, srcs, dsts, zrows, None, None,
                     out_acc, None, src_v, dst_v, rows_a, rows_b, ones_v,
                     acc_sh, deg_sh, sem, semb, semc, semd, seme, semf)

    return pl.kernel(body, out_type=out_type, mesh=mesh,
                     scratch_types=scratch)


BM = 1000


def _pre_body(x_ref, wl_ref, wr_ref, bl_ref, a_ref, r_ref):
    xb = x_ref[...]
    a_ref[...] = jnp.dot(xb, wl_ref[...], preferred_element_type=jnp.float32)
    r_ref[...] = (jnp.dot(xb, wr_ref[...], preferred_element_type=jnp.float32)
                  + bl_ref[...])


def _mid_body(sa_ref, sb_ref, da_ref, db_ref, r_ref, wl_ref, wr_ref, bl_ref,
              a_ref, rn_ref):
    invd = 1.0 / jnp.maximum(da_ref[...] + db_ref[...], 1.0)
    h = jnp.maximum((sa_ref[0] + sb_ref[0]) * invd + r_ref[...], 0.0)
    a_ref[...] = jnp.dot(h, wl_ref[...], preferred_element_type=jnp.float32)
    rn_ref[...] = (jnp.dot(h, wr_ref[...], preferred_element_type=jnp.float32)
                   + bl_ref[...])


def _mid2_body(sa_ref, sb_ref, da_ref, db_ref, r_ref, wr_ref, bl_ref,
               h_ref, rn_ref):
    invd = 1.0 / jnp.maximum(da_ref[...] + db_ref[...], 1.0)
    h = jnp.maximum((sa_ref[0] + sb_ref[0]) * invd + r_ref[...], 0.0)
    h_ref[...] = h
    rn_ref[...] = (jnp.dot(h, wr_ref[...], preferred_element_type=jnp.float32)
                   + bl_ref[...])


def _fin_body(sa_ref, sb_ref, da_ref, db_ref, r_ref, wl_ref, o_ref):
    invd = 1.0 / jnp.maximum(da_ref[...] + db_ref[...], 1.0)
    mean = (sa_ref[0] + sb_ref[0]) * invd
    o_ref[...] = (jnp.dot(mean, wl_ref[...], preferred_element_type=jnp.float32)
                  + r_ref[...])


def _row_spec(dcol):
    return pl.BlockSpec((BM, dcol), lambda i: (i, 0))


def _part_spec(dcol, core):
    return pl.BlockSpec((1, BM, dcol), lambda i, c=core: (c, i, 0))


def _full_spec(r, c):
    return pl.BlockSpec((r, c), lambda i: (0, 0))


def _tc_pre(x, wlT, wrT, bl, dout):
    din = x.shape[1]
    return pl.pallas_call(
        _pre_body,
        grid=(N // BM,),
        in_specs=[_row_spec(din), _full_spec(din, dout), _full_spec(din, dout),
                  _full_spec(1, dout)],
        out_specs=[_row_spec(dout), _row_spec(dout)],
        out_shape=[jax.ShapeDtypeStruct((N, dout), jnp.float32)] * 2,
    )(x, wlT, wrT, bl)


def _tc_mid(s2c, da, db, r, wlT, wrT, bl, dout):
    din = s2c.shape[2]
    sa = sb = s2c
    return pl.pallas_call(
        _mid_body,
        grid=(N // BM,),
        in_specs=[_part_spec(din, 0), _part_spec(din, 1), _row_spec(1),
                  _row_spec(1), _row_spec(din), _full_spec(din, dout),
                  _full_spec(din, dout), _full_spec(1, dout)],
        out_specs=[_row_spec(dout), _row_spec(dout)],
        out_shape=[jax.ShapeDtypeStruct((N, dout), jnp.float32)] * 2,
    )(sa, sb, da, db, r, wlT, wrT, bl)


def _tc_mid2(s2c, da, db, r, wrT, bl, dout):
    din = s2c.shape[2]
    sa = sb = s2c
    return pl.pallas_call(
        _mid2_body,
        grid=(N // BM,),
        in_specs=[_part_spec(din, 0), _part_spec(din, 1), _row_spec(1),
                  _row_spec(1), _row_spec(din), _full_spec(din, dout),
                  _full_spec(1, dout)],
        out_specs=[_row_spec(din), _row_spec(dout)],
        out_shape=[jax.ShapeDtypeStruct((N, din), jnp.float32),
                   jax.ShapeDtypeStruct((N, dout), jnp.float32)],
    )(sa, sb, da, db, r, wrT, bl)


def _tc_fin(s2c, da, db, r, wlT, dout):
    din = s2c.shape[2]
    sa = sb = s2c
    return pl.pallas_call(
        _fin_body,
        grid=(N // BM,),
        in_specs=[_part_spec(din, 0), _part_spec(din, 1), _row_spec(1),
                  _row_spec(1), _row_spec(dout), _full_spec(din, dout)],
        out_specs=_row_spec(dout),
        out_shape=jax.ShapeDtypeStruct((N, dout), jnp.float32),
    )(sa, sb, da, db, r, wlT)


@jax.jit
def kernel(x, edge_index, Wl0, bl0, Wr0, Wl1, bl1, Wr1, Wl2, bl2, Wr2):
    src = edge_index[0]
    dst = edge_index[1]
    pad = EPAD - E
    srcs = jnp.concatenate([src, jnp.zeros((pad,), jnp.int32)])
    dsts = jnp.concatenate([dst, jnp.full((pad,), NP - 1, jnp.int32)])
    srcs = srcs.reshape(NW, NCH, CHUNK)
    dsts = dsts.reshape(NW, NCH, CHUNK)

    zrows128 = jnp.zeros((RPT, D_HID), jnp.float32)
    zdeg = jnp.zeros((RPT,), jnp.float32)
    ones_in = jnp.ones((CHUNK,), jnp.float32)

    sc0 = _make_sc_scatter(D_HID, True)
    sc1 = _make_sc_scatter(D_HID, False)

    a0, r0 = _tc_pre(x, Wl0.T, Wr0.T, bl0[None, :], D_HID)
    s0, deg = sc0(a0, srcs, dsts, zrows128, zdeg, ones_in)
    da = deg[0, :N, None]
    db = deg[1, :N, None]

    a1, r1 = _tc_mid(s0, da, db, r0, Wl1.T, Wr1.T, bl1[None, :], D_HID)
    (s1,) = sc1(a1, srcs, dsts, zrows128)

    h2, r2 = _tc_mid2(s1, da, db, r1, Wr2.T, bl2[None, :], D_OUT)
    (s2,) = sc1(h2, srcs, dsts, zrows128)

    return _tc_fin(s2, da, db, r2, Wl2.T, D_OUT)
